# Initial kernel scaffold; baseline (speedup 1.0000x reference)
import jax
import jax.numpy as jnp
from jax.experimental import pallas as pl


def kernel(x, coefficients_vect):
    raise NotImplementedError("write your pallas kernel here")


# speedup vs baseline: 747.7201x; 747.7201x over previous
import functools

import jax
import jax.numpy as jnp
import numpy as np
from jax import lax
from jax.experimental import pallas as pl
from jax.experimental.pallas import tpu as pltpu
from jax.experimental.pallas import tpu_sc as plsc

_SIZE = 4096
_GRID = np.float32(0.001)
_HALF = _SIZE // 2
_CLIP_LO = np.float32(-(_GRID * np.float32(_HALF)))
_CLIP_HI = np.float32(_GRID * np.float32(_HALF - 1))

_NC = 2
_NS = 16
_NW = _NC * _NS
_LANES = 16

_CHUNK = 16384


def _body(x_hbm, tab_hbm, out_hbm, tab_v, in_v, out_v):
    wid = lax.axis_index("s") * _NC + lax.axis_index("c")
    per_w = x_hbm.shape[0] // _NW
    base = wid * per_w
    nchunks = per_w // _CHUNK
    nsteps = _CHUNK // _LANES

    pltpu.sync_copy(tab_hbm, tab_v)

    def chunk_body(ci, _):
        off = base + ci * _CHUNK
        pltpu.sync_copy(x_hbm.at[pl.ds(off, _CHUNK)], in_v)

        def step(i, _):
            xs = in_v[pl.ds(i * _LANES, _LANES)]
            xc = jnp.minimum(jnp.maximum(xs, _CLIP_LO), _CLIP_HI)
            q = xc / _GRID
            t = q.astype(jnp.int32)
            tf = t.astype(jnp.float32)
            fl = jnp.where(q < tf, t - 1, t)
            frac = q - fl.astype(jnp.float32)
            j = jnp.bitwise_and(fl, _SIZE - 1)
            c_lo = plsc.load_gather(tab_v, [j])
            c_hi = plsc.load_gather(tab_v, [j + 1])
            out_v[pl.ds(i * _LANES, _LANES)] = (
                c_hi * frac + c_lo * (np.float32(1.0) - frac)
            )
            return ()

        lax.fori_loop(0, nsteps, step, ())
        pltpu.sync_copy(out_v, out_hbm.at[pl.ds(off, _CHUNK)])
        return ()

    lax.fori_loop(0, nchunks, chunk_body, ())


@jax.jit
def _run(x_flat, table):
    n = x_flat.shape[0]
    mesh = plsc.VectorSubcoreMesh(core_axis_name="c", subcore_axis_name="s")
    k = functools.partial(
        pl.kernel,
        out_type=jax.ShapeDtypeStruct((n,), jnp.float32),
        mesh=mesh,
        scratch_types=[
            pltpu.VMEM((_SIZE + 1,), jnp.float32),
            pltpu.VMEM((_CHUNK,), jnp.float32),
            pltpu.VMEM((_CHUNK,), jnp.float32),
        ],
        compiler_params=pltpu.CompilerParams(needs_layout_passes=False),
    )(_body)
    return k(x_flat, table)


def kernel(x, coefficients_vect):
    table = jnp.concatenate([coefficients_vect, coefficients_vect[:1]])
    x_flat = x.reshape(-1)
    out = _run(x_flat, table)
    return out.reshape(x.shape)
